# baseline (speedup 1.0000x reference)
import functools

import jax
import jax.numpy as jnp
from jax import lax
from jax.experimental import pallas as pl
from jax.experimental.pallas import tpu as pltpu
from jax.experimental.pallas import tpu_sc as plsc

D_MODEL = 64
SCALE = 8.0
NUM_CORES = 2
NUM_SUBCORES = 16
NUM_WORKERS = NUM_CORES * NUM_SUBCORES
CHUNK = 512
ROWS_PER_ITER = 4


@functools.lru_cache(maxsize=None)
def _make_kernel(B: int):
    b_per_w = B // NUM_WORKERS
    n_chunks = b_per_w // CHUNK
    mesh = plsc.VectorSubcoreMesh(core_axis_name="c", subcore_axis_name="s")

    @functools.partial(
        pl.kernel,
        mesh=mesh,
        out_type=jax.ShapeDtypeStruct((B, D_MODEL), jnp.float32),
        scratch_types=[
            pltpu.VMEM((CHUNK,), jnp.int32),
            pltpu.VMEM((CHUNK, D_MODEL), jnp.float32),
            pltpu.SemaphoreType.DMA,
        ],
        compiler_params=pltpu.CompilerParams(use_tc_tiling_on_sc=False),
    )
    def emb(x_hbm, table_hbm, out_hbm, idx_v, rows_v, sem):
        wid = lax.axis_index("s") * NUM_CORES + lax.axis_index("c")
        base = wid * b_per_w

        def chunk_body(c, carry):
            off = pl.multiple_of(base + c * CHUNK, 8)
            pltpu.sync_copy(x_hbm.at[pl.ds(off, CHUNK)], idx_v)
            pltpu.async_copy(table_hbm.at[idx_v], rows_v, sem).wait()

            def scale_body(i, carry2):
                for u in range(ROWS_PER_ITER):
                    r = i * ROWS_PER_ITER + u
                    for j in range(D_MODEL // 16):
                        sl = (r, pl.ds(j * 16, 16))
                        rows_v[sl] = rows_v[sl] * SCALE
                return carry2

            lax.fori_loop(0, CHUNK // ROWS_PER_ITER, scale_body, 0)
            pltpu.sync_copy(rows_v, out_hbm.at[pl.ds(off, CHUNK)])
            return carry

        lax.fori_loop(0, n_chunks, chunk_body, 0)

    return emb


def kernel(x, table):
    B = x.size
    out = _make_kernel(B)(x.reshape(-1), table)
    return out.reshape(*x.shape, D_MODEL)
